# baseline (speedup 1.0000x reference)
import functools

import jax
import jax.numpy as jnp
from jax import lax
from jax.experimental import pallas as pl
from jax.experimental.pallas import tpu as pltpu
from jax.experimental.pallas import tpu_sc as plsc

LANES = 16
NUM_WORKERS = 32
SUB = 256
KSUB = 1
BLOCK = SUB * KSUB
NODE_CHUNK = 128


def _sc_body(x_hbm, h_hbm, j_hbm, ii_hbm, ij_hbm, out_hbm, *scratch):
    (ii0, ii1, ij0, ij1, jv0, jv1, xi0, xi1, xj0, xj1,
     hx_v, h_v, out_v, x_sp, sl0, sl1, sg0, sg1, sn) = scratch
    slots = ((ii0, ij0, jv0, xi0, xj0, sl0, sg0),
             (ii1, ij1, jv1, xi1, xj1, sl1, sg1))

    wid = lax.axis_index("s") * 2 + lax.axis_index("c")
    n_total = x_hbm.shape[0]
    m_total = j_hbm.shape[0]
    nodes_pw = n_total // NUM_WORKERS
    edges_pw = m_total // NUM_WORKERS
    node_chunks = nodes_pw // NODE_CHUNK
    nb = edges_pw // BLOCK
    node_base = wid * nodes_pw
    edge_base = wid * edges_pw
    row_base = edge_base // SUB

    sid = lax.axis_index("s")
    rows_per_sub = n_total // 16
    soff = sid * rows_per_sub
    pltpu.async_copy(x_hbm.at[pl.ds(soff, rows_per_sub)],
                     x_sp.at[pl.ds(soff, rows_per_sub)], sn).wait()
    plsc.subcore_barrier()

    def node_chunk_body(c, acc):
        off = node_base + c * NODE_CHUNK
        cp0 = pltpu.async_copy(x_hbm.at[pl.ds(off, NODE_CHUNK)], hx_v, sn)
        cp1 = pltpu.async_copy(h_hbm.at[pl.ds(off, NODE_CHUNK)], h_v, sn)
        cp0.wait()
        cp1.wait()

        def n_body(g, a):
            hv = h_v[pl.ds(g * LANES, LANES)]
            for k in range(LANES):
                a = a + hv[k] * hx_v[g * LANES + k, :]
            return a

        return lax.fori_loop(0, NODE_CHUNK // LANES, n_body, acc)

    acc = lax.fori_loop(0, node_chunks, node_chunk_body,
                        jnp.zeros((LANES,), jnp.float32))

    def lin_descrs(g, slot):
        ii_v, ij_v, jv_v, _, _, sl, _ = slots[slot]
        roff = row_base + g * KSUB
        eoff = edge_base + g * BLOCK
        return (pltpu.make_async_copy(ii_hbm.at[pl.ds(roff, KSUB)], ii_v, sl),
                pltpu.make_async_copy(ij_hbm.at[pl.ds(roff, KSUB)], ij_v, sl),
                pltpu.make_async_copy(j_hbm.at[pl.ds(eoff, BLOCK)], jv_v, sl))

    def gat_descrs(slot):
        ii_v, ij_v, _, xi_v, xj_v, _, sg = slots[slot]
        ds = []
        for k in range(KSUB):
            dst_i = xi_v.at[pl.ds(k * SUB, SUB)]
            dst_j = xj_v.at[pl.ds(k * SUB, SUB)]
            ds.append(pltpu.make_async_copy(x_sp.at[ii_v.at[k]], dst_i, sg))
            ds.append(pltpu.make_async_copy(x_sp.at[ij_v.at[k]], dst_j, sg))
        return ds

    def issue_lin(g, slot):
        for d in lin_descrs(g, slot):
            d.start()

    def wait_lin(g, slot):
        for d in lin_descrs(g, slot):
            d.wait()

    def issue_gat(slot):
        for d in gat_descrs(slot):
            d.start()

    def wait_gat(slot):
        for d in gat_descrs(slot):
            d.wait()

    def blk_compute(slot, acc):
        _, _, jv_v, xi_v, xj_v, _, _ = slots[slot]

        def e_body(g2, a):
            jv = jv_v[pl.ds(g2 * LANES, LANES)]
            for k in range(LANES):
                e = g2 * LANES + k
                a = a + jv[k] * (xi_v[e, :] * xj_v[e, :])
            return a

        return lax.fori_loop(0, BLOCK // LANES, e_body, acc)

    issue_lin(0, 0)
    wait_lin(0, 0)
    issue_gat(0)
    issue_lin(1, 1)

    def pair_body(p, acc):
        g0 = 2 * p
        wait_gat(0)
        wait_lin(g0 + 1, 1)
        issue_gat(1)
        acc = blk_compute(0, acc)

        @pl.when(g0 + 2 < nb)
        def _():
            issue_lin(g0 + 2, 0)

        wait_gat(1)

        @pl.when(g0 + 2 < nb)
        def _():
            wait_lin(g0 + 2, 0)
            issue_gat(0)

        acc = blk_compute(1, acc)

        @pl.when(g0 + 3 < nb)
        def _():
            issue_lin(g0 + 3, 1)

        return acc

    acc = lax.fori_loop(0, nb // 2, pair_body, acc)

    out_v[:] = acc
    pltpu.async_copy(out_v, out_hbm.at[wid], sn).wait()


def _round_up(v, m):
    return (v + m - 1) // m * m


@jax.jit
def _run(x_t, h_p, j_p, ii_p, ij_p):
    run = pl.kernel(
        _sc_body,
        out_type=jax.ShapeDtypeStruct((NUM_WORKERS, LANES), jnp.float32),
        mesh=plsc.VectorSubcoreMesh(core_axis_name="c", subcore_axis_name="s"),
        compiler_params=pltpu.CompilerParams(use_tc_tiling_on_sc=False),
        scratch_types=[
            pltpu.VMEM((KSUB, SUB), jnp.int32),
            pltpu.VMEM((KSUB, SUB), jnp.int32),
            pltpu.VMEM((KSUB, SUB), jnp.int32),
            pltpu.VMEM((KSUB, SUB), jnp.int32),
            pltpu.VMEM((BLOCK,), jnp.float32),
            pltpu.VMEM((BLOCK,), jnp.float32),
            pltpu.VMEM((BLOCK, LANES), jnp.float32),
            pltpu.VMEM((BLOCK, LANES), jnp.float32),
            pltpu.VMEM((BLOCK, LANES), jnp.float32),
            pltpu.VMEM((BLOCK, LANES), jnp.float32),
            pltpu.VMEM((NODE_CHUNK, LANES), jnp.float32),
            pltpu.VMEM((NODE_CHUNK,), jnp.float32),
            pltpu.VMEM((LANES,), jnp.float32),
            pltpu.VMEM_SHARED((x_t.shape[0], LANES), jnp.float32),
            pltpu.SemaphoreType.DMA,
            pltpu.SemaphoreType.DMA,
            pltpu.SemaphoreType.DMA,
            pltpu.SemaphoreType.DMA,
            pltpu.SemaphoreType.DMA,
        ],
    )
    partials = run(x_t, h_p, j_p, ii_p, ij_p)
    return partials.sum(axis=0)


def kernel(x, h, J, edge_idx_i, edge_idx_j):
    B, N = x.shape
    M = J.shape[0]
    assert B == LANES
    NP = _round_up(N, NUM_WORKERS * NODE_CHUNK)
    MP = _round_up(M, NUM_WORKERS * BLOCK * 2)
    x_t = jnp.zeros((NP, B), jnp.float32).at[:N].set(x.T)
    h_p = jnp.zeros((NP,), jnp.float32).at[:N].set(h)
    j_p = jnp.zeros((MP,), jnp.float32).at[:M].set(J)
    ii_p = jnp.zeros((MP,), jnp.int32).at[:M].set(edge_idx_i).reshape(MP // SUB, SUB)
    ij_p = jnp.zeros((MP,), jnp.int32).at[:M].set(edge_idx_j).reshape(MP // SUB, SUB)
    return _run(x_t, h_p, j_p, ii_p, ij_p)


# speedup vs baseline: 1.2549x; 1.2549x over previous
import functools

import jax
import jax.numpy as jnp
import numpy as np
from jax import lax
from jax.experimental import pallas as pl
from jax.experimental.pallas import tpu as pltpu
from jax.experimental.pallas import tpu_sc as plsc

LANES = 16
NUM_WORKERS = 32
SUB = 200
KSUB = 2
BLOCK = SUB * KSUB
NODE_CHUNK = BLOCK


def _sc_body(x_hbm, h_hbm, j_hbm, ii_hbm, ij_hbm, out_hbm, *scratch):
    (ii0, ii1, ij0, ij1, jv0, jv1, xi0, xi1, xj0, xj1,
     out_v, x_sp, sl0, sl1, sg0, sg1, sn) = scratch
    slots = ((ii0, ij0, jv0, xi0, xj0, sl0, sg0),
             (ii1, ij1, jv1, xi1, xj1, sl1, sg1))

    wid = lax.axis_index("s") * 2 + lax.axis_index("c")
    np_total = x_hbm.shape[0]
    n_stage = x_sp.shape[0]
    m_total = ii_hbm.shape[0] * SUB
    nodes_pw = np_total // NUM_WORKERS
    node_chunks = nodes_pw // NODE_CHUNK
    nb = m_total // (NUM_WORKERS * BLOCK)
    node_base = wid * nodes_pw

    sid = lax.axis_index("s")
    rows_per_sub = n_stage // 16
    soff = sid * rows_per_sub
    pltpu.async_copy(x_hbm.at[pl.ds(soff, rows_per_sub)],
                     x_sp.at[pl.ds(soff, rows_per_sub)], sn).wait()

    def node_chunk_body(c, acc):
        off = node_base + c * NODE_CHUNK
        cp0 = pltpu.async_copy(x_hbm.at[pl.ds(off, NODE_CHUNK)], xi0, sn)
        cp1 = pltpu.async_copy(
            h_hbm.at[pl.ds(off // NODE_CHUNK, 1)], jv0, sn)
        cp0.wait()
        cp1.wait()

        def n_body(g, a):
            hv = jv0[0, pl.ds(g * LANES, LANES)]
            for k in range(LANES):
                a = a + hv[k] * xi0[g * LANES + k, :]
            return a

        return lax.fori_loop(0, NODE_CHUNK // LANES, n_body, acc)

    acc = lax.fori_loop(0, node_chunks, node_chunk_body,
                        jnp.zeros((LANES,), jnp.float32))

    plsc.subcore_barrier()

    def lin_descrs(g, slot):
        ii_v, ij_v, jv_v, _, _, sl, _ = slots[slot]
        bid = wid * nb + g
        return (pltpu.make_async_copy(ii_hbm.at[pl.ds(bid * KSUB, KSUB)],
                                      ii_v, sl),
                pltpu.make_async_copy(ij_hbm.at[pl.ds(bid * KSUB, KSUB)],
                                      ij_v, sl),
                pltpu.make_async_copy(j_hbm.at[pl.ds(bid, 1)], jv_v, sl))

    def gat_descrs(slot):
        ii_v, ij_v, _, xi_v, xj_v, _, sg = slots[slot]
        ds = []
        for k in range(KSUB):
            dst_i = xi_v.at[pl.ds(k * SUB, SUB)]
            dst_j = xj_v.at[pl.ds(k * SUB, SUB)]
            ds.append(pltpu.make_async_copy(x_sp.at[ii_v.at[k]], dst_i, sg))
            ds.append(pltpu.make_async_copy(x_sp.at[ij_v.at[k]], dst_j, sg))
        return ds

    def issue_lin(g, slot):
        for d in lin_descrs(g, slot):
            d.start()

    def wait_lin(g, slot):
        for d in lin_descrs(g, slot):
            d.wait()

    def issue_gat(slot):
        for d in gat_descrs(slot):
            d.start()

    def wait_gat(slot):
        for d in gat_descrs(slot):
            d.wait()

    def blk_compute(slot, acc):
        _, _, jv_v, xi_v, xj_v, _, _ = slots[slot]

        def e_body(g2, a):
            jv = jv_v[0, pl.ds(g2 * LANES, LANES)]
            for k in range(LANES):
                e = g2 * LANES + k
                a = a + jv[k] * (xi_v[e, :] * xj_v[e, :])
            return a

        return lax.fori_loop(0, BLOCK // LANES, e_body, acc)

    issue_lin(0, 0)
    wait_lin(0, 0)
    issue_gat(0)
    issue_lin(1, 1)

    def pair_body(p, acc):
        g0 = 2 * p
        wait_gat(0)
        wait_lin(g0 + 1, 1)
        issue_gat(1)
        acc = blk_compute(0, acc)

        @pl.when(g0 + 2 < nb)
        def _():
            issue_lin(g0 + 2, 0)

        wait_gat(1)

        @pl.when(g0 + 2 < nb)
        def _():
            wait_lin(g0 + 2, 0)
            issue_gat(0)

        acc = blk_compute(1, acc)

        @pl.when(g0 + 3 < nb)
        def _():
            issue_lin(g0 + 3, 1)

        return acc

    acc = lax.fori_loop(0, nb // 2, pair_body, acc)

    out_v[:] = acc
    pltpu.async_copy(out_v, out_hbm.at[wid], sn).wait()


def _round_up(v, m):
    return (v + m - 1) // m * m


@functools.partial(jax.jit, static_argnums=(5,))
def _run(x_t, h_p, j_p, ii_p, ij_p, n_stage):
    run = pl.kernel(
        _sc_body,
        out_type=jax.ShapeDtypeStruct((NUM_WORKERS, LANES), jnp.float32),
        mesh=plsc.VectorSubcoreMesh(core_axis_name="c", subcore_axis_name="s"),
        compiler_params=pltpu.CompilerParams(use_tc_tiling_on_sc=False),
        scratch_types=[
            pltpu.VMEM((KSUB, SUB), jnp.int32),
            pltpu.VMEM((KSUB, SUB), jnp.int32),
            pltpu.VMEM((KSUB, SUB), jnp.int32),
            pltpu.VMEM((KSUB, SUB), jnp.int32),
            pltpu.VMEM((1, BLOCK), jnp.float32),
            pltpu.VMEM((1, BLOCK), jnp.float32),
            pltpu.VMEM((BLOCK, LANES), jnp.float32),
            pltpu.VMEM((BLOCK, LANES), jnp.float32),
            pltpu.VMEM((BLOCK, LANES), jnp.float32),
            pltpu.VMEM((BLOCK, LANES), jnp.float32),
            pltpu.VMEM((LANES,), jnp.float32),
            pltpu.VMEM_SHARED((n_stage, LANES), jnp.float32),
            pltpu.SemaphoreType.DMA,
            pltpu.SemaphoreType.DMA,
            pltpu.SemaphoreType.DMA,
            pltpu.SemaphoreType.DMA,
            pltpu.SemaphoreType.DMA,
        ],
    )
    partials = run(x_t, h_p, j_p, ii_p, ij_p)
    return partials.sum(axis=0)


def kernel(x, h, J, edge_idx_i, edge_idx_j):
    B, N = x.shape
    M = J.shape[0]
    assert B == LANES
    NP = _round_up(N, NUM_WORKERS * NODE_CHUNK)
    NSTAGE = _round_up(N, 16)
    MP = _round_up(M, NUM_WORKERS * BLOCK * 2)
    x_t = jnp.zeros((NP, B), jnp.float32).at[:N].set(x.T)
    h_p = jnp.zeros((NP,), jnp.float32).at[:N].set(h)
    if MP != M:
        J = jnp.zeros((MP,), jnp.float32).at[:M].set(J)
        edge_idx_i = jnp.zeros((MP,), jnp.int32).at[:M].set(edge_idx_i)
        edge_idx_j = jnp.zeros((MP,), jnp.int32).at[:M].set(edge_idx_j)
    h_2d = h_p.reshape(NP // NODE_CHUNK, NODE_CHUNK)
    j_2d = J.reshape(MP // BLOCK, BLOCK)
    ii_2d = edge_idx_i.reshape(MP // SUB, SUB)
    ij_2d = edge_idx_j.reshape(MP // SUB, SUB)
    return _run(x_t, h_2d, j_2d, ii_2d, ij_2d, NSTAGE)


# speedup vs baseline: 1.2591x; 1.0033x over previous
import functools

import jax
import jax.numpy as jnp
import numpy as np
from jax import lax
from jax.experimental import pallas as pl
from jax.experimental.pallas import tpu as pltpu
from jax.experimental.pallas import tpu_sc as plsc

LANES = 16
NUM_WORKERS = 32
SUB = 400
KSUB = 1
BLOCK = SUB * KSUB
NODE_CHUNK = BLOCK


def _sc_body(x_hbm, h_hbm, j_hbm, ii_hbm, ij_hbm, out_hbm, *scratch):
    (ii0, ii1, ij0, ij1, jv0, jv1, xi0, xi1, xj0, xj1,
     out_v, x_sp, sl0, sl1, sg0, sg1, sn) = scratch
    slots = ((ii0, ij0, jv0, xi0, xj0, sl0, sg0),
             (ii1, ij1, jv1, xi1, xj1, sl1, sg1))

    wid = lax.axis_index("s") * 2 + lax.axis_index("c")
    np_total = x_hbm.shape[0]
    n_stage = x_sp.shape[0]
    m_total = ii_hbm.shape[0] * SUB
    nodes_pw = np_total // NUM_WORKERS
    node_chunks = nodes_pw // NODE_CHUNK
    nb = m_total // (NUM_WORKERS * BLOCK)
    node_base = wid * nodes_pw

    sid = lax.axis_index("s")
    rows_per_sub = n_stage // 16
    soff = sid * rows_per_sub
    pltpu.async_copy(x_hbm.at[pl.ds(soff, rows_per_sub)],
                     x_sp.at[pl.ds(soff, rows_per_sub)], sn).wait()

    def node_chunk_body(c, acc):
        off = node_base + c * NODE_CHUNK
        cp0 = pltpu.async_copy(x_hbm.at[pl.ds(off, NODE_CHUNK)], xi0, sn)
        cp1 = pltpu.async_copy(
            h_hbm.at[pl.ds(off // NODE_CHUNK, 1)], jv0, sn)
        cp0.wait()
        cp1.wait()

        def n_body(g, a):
            hv = jv0[0, pl.ds(g * LANES, LANES)]
            for k in range(LANES):
                a = a + hv[k] * xi0[g * LANES + k, :]
            return a

        return lax.fori_loop(0, NODE_CHUNK // LANES, n_body, acc)

    acc = lax.fori_loop(0, node_chunks, node_chunk_body,
                        jnp.zeros((LANES,), jnp.float32))

    plsc.subcore_barrier()

    def lin_descrs(g, slot):
        ii_v, ij_v, jv_v, _, _, sl, _ = slots[slot]
        bid = wid * nb + g
        return (pltpu.make_async_copy(ii_hbm.at[pl.ds(bid * KSUB, KSUB)],
                                      ii_v, sl),
                pltpu.make_async_copy(ij_hbm.at[pl.ds(bid * KSUB, KSUB)],
                                      ij_v, sl),
                pltpu.make_async_copy(j_hbm.at[pl.ds(bid, 1)], jv_v, sl))

    def gat_descrs(slot):
        ii_v, ij_v, _, xi_v, xj_v, _, sg = slots[slot]
        ds = []
        for k in range(KSUB):
            dst_i = xi_v.at[pl.ds(k * SUB, SUB)]
            dst_j = xj_v.at[pl.ds(k * SUB, SUB)]
            ds.append(pltpu.make_async_copy(x_sp.at[ii_v.at[k]], dst_i, sg))
            ds.append(pltpu.make_async_copy(x_sp.at[ij_v.at[k]], dst_j, sg))
        return ds

    def issue_lin(g, slot):
        for d in lin_descrs(g, slot):
            d.start()

    def wait_lin(g, slot):
        for d in lin_descrs(g, slot):
            d.wait()

    def issue_gat(slot):
        for d in gat_descrs(slot):
            d.start()

    def wait_gat(slot):
        for d in gat_descrs(slot):
            d.wait()

    def blk_compute(slot, acc):
        _, _, jv_v, xi_v, xj_v, _, _ = slots[slot]

        def e_body(g2, a):
            jv = jv_v[0, pl.ds(g2 * LANES, LANES)]
            for k in range(LANES):
                e = g2 * LANES + k
                a = a + jv[k] * (xi_v[e, :] * xj_v[e, :])
            return a

        return lax.fori_loop(0, BLOCK // LANES, e_body, acc)

    issue_lin(0, 0)
    wait_lin(0, 0)
    issue_gat(0)
    issue_lin(1, 1)

    def pair_body(p, acc):
        g0 = 2 * p
        wait_gat(0)
        wait_lin(g0 + 1, 1)
        issue_gat(1)
        acc = blk_compute(0, acc)

        @pl.when(g0 + 2 < nb)
        def _():
            issue_lin(g0 + 2, 0)

        wait_gat(1)

        @pl.when(g0 + 2 < nb)
        def _():
            wait_lin(g0 + 2, 0)
            issue_gat(0)

        acc = blk_compute(1, acc)

        @pl.when(g0 + 3 < nb)
        def _():
            issue_lin(g0 + 3, 1)

        return acc

    acc = lax.fori_loop(0, nb // 2, pair_body, acc)

    out_v[:] = acc
    pltpu.async_copy(out_v, out_hbm.at[wid], sn).wait()


def _round_up(v, m):
    return (v + m - 1) // m * m


@functools.partial(jax.jit, static_argnums=(5,))
def _run(x_t, h_p, j_p, ii_p, ij_p, n_stage):
    run = pl.kernel(
        _sc_body,
        out_type=jax.ShapeDtypeStruct((NUM_WORKERS, LANES), jnp.float32),
        mesh=plsc.VectorSubcoreMesh(core_axis_name="c", subcore_axis_name="s"),
        compiler_params=pltpu.CompilerParams(use_tc_tiling_on_sc=False),
        scratch_types=[
            pltpu.VMEM((KSUB, SUB), jnp.int32),
            pltpu.VMEM((KSUB, SUB), jnp.int32),
            pltpu.VMEM((KSUB, SUB), jnp.int32),
            pltpu.VMEM((KSUB, SUB), jnp.int32),
            pltpu.VMEM((1, BLOCK), jnp.float32),
            pltpu.VMEM((1, BLOCK), jnp.float32),
            pltpu.VMEM((BLOCK, LANES), jnp.float32),
            pltpu.VMEM((BLOCK, LANES), jnp.float32),
            pltpu.VMEM((BLOCK, LANES), jnp.float32),
            pltpu.VMEM((BLOCK, LANES), jnp.float32),
            pltpu.VMEM((LANES,), jnp.float32),
            pltpu.VMEM_SHARED((n_stage, LANES), jnp.float32),
            pltpu.SemaphoreType.DMA,
            pltpu.SemaphoreType.DMA,
            pltpu.SemaphoreType.DMA,
            pltpu.SemaphoreType.DMA,
            pltpu.SemaphoreType.DMA,
        ],
    )
    partials = run(x_t, h_p, j_p, ii_p, ij_p)
    return partials.sum(axis=0)


def kernel(x, h, J, edge_idx_i, edge_idx_j):
    B, N = x.shape
    M = J.shape[0]
    assert B == LANES
    NP = _round_up(N, NUM_WORKERS * NODE_CHUNK)
    NSTAGE = _round_up(N, 16)
    MP = _round_up(M, NUM_WORKERS * BLOCK * 2)
    x_t = jnp.zeros((NP, B), jnp.float32).at[:N].set(x.T)
    h_p = jnp.zeros((NP,), jnp.float32).at[:N].set(h)
    if MP != M:
        J = jnp.zeros((MP,), jnp.float32).at[:M].set(J)
        edge_idx_i = jnp.zeros((MP,), jnp.int32).at[:M].set(edge_idx_i)
        edge_idx_j = jnp.zeros((MP,), jnp.int32).at[:M].set(edge_idx_j)
    h_2d = h_p.reshape(NP // NODE_CHUNK, NODE_CHUNK)
    j_2d = J.reshape(MP // BLOCK, BLOCK)
    ii_2d = edge_idx_i.reshape(MP // SUB, SUB)
    ij_2d = edge_idx_j.reshape(MP // SUB, SUB)
    return _run(x_t, h_2d, j_2d, ii_2d, ij_2d, NSTAGE)
